# baseline (speedup 1.0000x reference)
import dataclasses
import functools

import jax
import jax.numpy as jnp
from jax import lax
from jax.experimental import pallas as pl
from jax.experimental.pallas import tpu as pltpu
from jax.experimental.pallas import tpu_sc as plsc


def _body(cur_smem, nxt_smem, xv_ref, emb_ref, dummy_ref, out_ref, buf_ref,
          sem, *, L):
    t = pl.program_id(0)
    nt = pl.num_programs(0)
    TB = out_ref.shape[0]

    def issue(idx_smem, slot):
        def issue_rows(b, carry):
            for l in range(L):
                idx = idx_smem[b, l]
                pltpu.make_async_copy(
                    emb_ref.at[idx], buf_ref.at[slot, b, l], sem.at[slot]
                ).start()
            return carry

        jax.lax.fori_loop(0, TB, issue_rows, 0)

    def drain(slot):
        def drain_rows(b, carry):
            for l in range(L):
                pltpu.make_async_copy(
                    dummy_ref.at[l], buf_ref.at[slot, b, l], sem.at[slot]
                ).wait()
            return carry

        jax.lax.fori_loop(0, TB, drain_rows, 0)

    parity = jax.lax.rem(t, 2)

    @pl.when(t == 0)
    def _():
        issue(cur_smem, 0)

    @pl.when(jnp.logical_and(t + 1 < nt, parity == 0))
    def _():
        issue(nxt_smem, 1)

    @pl.when(jnp.logical_and(t + 1 < nt, parity == 1))
    def _():
        issue(nxt_smem, 0)

    @pl.when(parity == 0)
    def _():
        drain(0)

    @pl.when(parity == 1)
    def _():
        drain(1)

    mask = (xv_ref[...] != 0).astype(jnp.float32)

    @pl.when(parity == 0)
    def _():
        out_ref[...] = jnp.tanh(jnp.sum(buf_ref[0] * mask, axis=1))

    @pl.when(parity == 1)
    def _():
        out_ref[...] = jnp.tanh(jnp.sum(buf_ref[1] * mask, axis=1))


def _phedvec(x, embeddings, tb, interpret=False):
    B, L = x.shape
    _, D = embeddings.shape
    nt = B // tb
    grid_spec = pltpu.PrefetchScalarGridSpec(
        num_scalar_prefetch=0,
        grid=(nt,),
        in_specs=[
            pl.BlockSpec((tb, L), lambda t: (t, 0), memory_space=pltpu.SMEM),
            pl.BlockSpec(
                (tb, L),
                lambda t: (jnp.minimum(t + 1, nt - 1), 0),
                memory_space=pltpu.SMEM,
            ),
            pl.BlockSpec((tb, L, 1), lambda t: (t, 0, 0)),
            pl.BlockSpec(memory_space=pltpu.HBM),
            pl.BlockSpec(memory_space=pltpu.HBM),
        ],
        out_specs=pl.BlockSpec((tb, D), lambda t: (t, 0)),
        scratch_shapes=[
            pltpu.VMEM((2, tb, L, D), jnp.float32),
            pltpu.SemaphoreType.DMA((2,)),
        ],
    )
    return pl.pallas_call(
        functools.partial(_body, L=L),
        grid_spec=grid_spec,
        out_shape=jax.ShapeDtypeStruct((B, D), jnp.float32),
        compiler_params=pltpu.CompilerParams(
            dimension_semantics=("arbitrary",),
        ),
        interpret=interpret,
    )(x, x, x.reshape(B, L, 1), embeddings,
      jnp.zeros((L, D), jnp.float32))


_NC = 2
_NS = 16
_NW = _NC * _NS
_CHUNK = 16


def _sc_sums(x, table3):
    B, L = x.shape
    _, SL, LN = table3.shape
    PW = B // _NW
    NQ = LN // _CHUNK
    mesh = plsc.VectorSubcoreMesh(core_axis_name="c", subcore_axis_name="s")

    @functools.partial(
        pl.kernel,
        out_type=jax.ShapeDtypeStruct((B, SL, LN), jnp.float32),
        mesh=mesh,
        scratch_types=[
            pltpu.VMEM((PW, L), jnp.int32),
            pltpu.VMEM((2, L, SL, LN), jnp.float32),
            pltpu.VMEM((2, 2, SL, LN), jnp.float32),
            pltpu.SemaphoreType.DMA((2,)),
            pltpu.SemaphoreType.DMA((2,)),
        ],
    )
    def sums_kernel(x_hbm, table_hbm, out_hbm, idx_v, rows_v, ostage, gsem,
                    osem):
        wid = lax.axis_index("s") * _NC + lax.axis_index("c")
        base = wid * PW

        pltpu.sync_copy(x_hbm.at[pl.ds(base, PW)], idx_v)

        def start_gather(j, slot):
            pltpu.make_async_copy(
                table_hbm.at[idx_v.at[j]], rows_v.at[slot], gsem.at[slot]
            ).start()

        def wait_gather(j, slot):
            pltpu.make_async_copy(
                table_hbm.at[idx_v.at[j]], rows_v.at[slot], gsem.at[slot]
            ).wait()

        def accum(rslot, oslot, orow):
            src = rows_v.at[rslot]

            @pl.loop(0, SL)
            def _(s):
                @pl.loop(0, NQ)
                def _(q):
                    off = q * _CHUNK
                    acc = jnp.zeros((_CHUNK,), jnp.float32)
                    for r in range(L):
                        acc = acc + src[r, s, pl.ds(off, _CHUNK)]
                    ostage[oslot, orow, s, pl.ds(off, _CHUNK)] = acc

        def start_out(oslot, j):
            pltpu.make_async_copy(
                ostage.at[oslot], out_hbm.at[pl.ds(base + j, 2)],
                osem.at[oslot],
            ).start()

        def wait_out(oslot, j):
            pltpu.make_async_copy(
                ostage.at[oslot], out_hbm.at[pl.ds(base + j, 2)],
                osem.at[oslot],
            ).wait()

        start_gather(0, 0)
        start_gather(1, 1)

        @pl.loop(0, PW, step=4)
        def _(j):
            @pl.when(j >= 4)
            def _():
                wait_out(0, j - 4)

            wait_gather(j, 0)
            accum(0, 0, 0)
            start_gather(j + 2, 0)
            wait_gather(j + 1, 1)
            accum(1, 0, 1)
            start_gather(j + 3, 1)
            start_out(0, j)

            @pl.when(j >= 4)
            def _():
                wait_out(1, j - 2)

            wait_gather(j + 2, 0)
            accum(0, 1, 0)

            @pl.when(j + 4 < PW)
            def _():
                start_gather(j + 4, 0)

            wait_gather(j + 3, 1)
            accum(1, 1, 1)

            @pl.when(j + 5 < PW)
            def _():
                start_gather(j + 5, 1)

            start_out(1, j + 2)

        wait_out(0, PW - 4)
        wait_out(1, PW - 2)

    return sums_kernel(x, table3)


def _fix_body(acc_ref, xv_ref, e0_ref, out_ref):
    D = out_ref.shape[1]
    n0 = jnp.sum((xv_ref[...] == 0).astype(jnp.float32), axis=1)
    fixed = jnp.tanh(acc_ref[...] - n0 * e0_ref[...])
    out_ref[...] = fixed[:, :D]


def _tanh_fix(sums, x3, e0p, d_out, tb):
    B, DP = sums.shape
    nt = B // tb
    return pl.pallas_call(
        _fix_body,
        grid=(nt,),
        in_specs=[
            pl.BlockSpec((tb, DP), lambda t: (t, 0)),
            pl.BlockSpec((tb, x3.shape[1], 1), lambda t: (t, 0, 0)),
            pl.BlockSpec((1, DP), lambda t: (0, 0)),
        ],
        out_specs=pl.BlockSpec((tb, d_out), lambda t: (t, 0)),
        out_shape=jax.ShapeDtypeStruct((B, d_out), jnp.float32),
        compiler_params=pltpu.CompilerParams(
            dimension_semantics=("arbitrary",),
        ),
    )(sums, x3, e0p)


def kernel(x, embeddings):
    xi = x.astype(jnp.int32)
    B, L = xi.shape
    V, D = embeddings.shape
    pad = (-D) % 128
    DP = D + pad
    tablep = jnp.pad(embeddings, ((0, 0), (0, pad)))
    sums = _sc_sums(xi, tablep.reshape(V, DP // 128, 128))
    e0p = lax.slice(tablep, (0, 0), (1, DP))
    return _tanh_fix(sums.reshape(B, DP), xi.reshape(B, L, 1), e0p, D,
                     tb=256)


# speedup vs baseline: 2.3451x; 2.3451x over previous
import functools

import jax
import jax.numpy as jnp
from jax import lax
from jax.experimental import pallas as pl
from jax.experimental.pallas import tpu as pltpu
from jax.experimental.pallas import tpu_sc as plsc

_NC = 2
_NS = 16
_NW = _NC * _NS
_CHUNK = 16
_LT = 128
_LS = 64


def _repack_body(in_ref, out_ref, *, D):
    SL = out_ref.shape[1]
    for g in range(SL):
        w = min(_LT, D - g * _LT)
        if w <= 0:
            break
        out_ref[:, g, :w] = in_ref[:, g * _LT:g * _LT + w]


def _repack_table(embeddings, sl):
    V, D = embeddings.shape
    R = 512
    nt = pl.cdiv(V, R)
    return pl.pallas_call(
        functools.partial(_repack_body, D=D),
        grid=(nt,),
        in_specs=[pl.BlockSpec((R, sl * _LT), lambda t: (t, 0))],
        out_specs=pl.BlockSpec((R, sl, _LT), lambda t: (t, 0, 0)),
        out_shape=jax.ShapeDtypeStruct((V, sl, _LT), jnp.float32),
        compiler_params=pltpu.CompilerParams(
            dimension_semantics=("arbitrary",),
        ),
    )(embeddings)


_IC = 32


def _sc_sums(x, table3, B, L):
    V, SL, LT = table3.shape
    PW = B // _NW
    NQ = _LT // _CHUNK
    mesh = plsc.VectorSubcoreMesh(core_axis_name="c", subcore_axis_name="s")

    @functools.partial(
        pl.kernel,
        out_type=jax.ShapeDtypeStruct((B, SL, _LT), jnp.float32),
        mesh=mesh,
        scratch_types=[
            pltpu.VMEM((2, _IC, L), jnp.int32),
            pltpu.VMEM((2, L, SL, _LT), jnp.float32),
            pltpu.VMEM((2, 2, SL, _LT), jnp.float32),
            pltpu.SemaphoreType.DMA((2,)),
            pltpu.SemaphoreType.DMA((2,)),
            pltpu.SemaphoreType.DMA,
        ],
    )
    def sums_kernel(x_hbm, table_hbm, out_hbm, idx_v, rows_v, ostage, gsem,
                    osem, isem):
        wid = lax.axis_index("s") * _NC + lax.axis_index("c")
        base = wid * PW

        def idx_copy(chunk):
            return pltpu.make_async_copy(
                x_hbm.at[pl.ds(base + chunk * _IC, _IC)],
                idx_v.at[jax.lax.rem(chunk, 2)],
                isem,
            )

        pltpu.sync_copy(x_hbm.at[pl.ds(base, _IC)], idx_v.at[0])
        idx_copy(1).start()

        def gather_copy(j, slot):
            jidx = idx_v.at[jax.lax.rem(j // _IC, 2), jax.lax.rem(j, _IC)]
            return pltpu.make_async_copy(
                table_hbm.at[jidx], rows_v.at[slot], gsem.at[slot]
            )

        def start_gather(j, slot):
            gather_copy(j, slot).start()

        def wait_gather(j, slot):
            gather_copy(j, slot).wait()

        def accum(rslot, oslot, orow):
            src = rows_v.at[rslot]

            @pl.loop(0, SL)
            def _(g):
                @pl.loop(0, NQ)
                def _(q):
                    acc = jnp.zeros((_CHUNK,), jnp.float32)
                    for r in range(L):
                        acc = acc + src[r, g, pl.ds(q * _CHUNK, _CHUNK)]
                    ostage[oslot, orow, g, pl.ds(q * _CHUNK, _CHUNK)] = acc

        def start_out(oslot, j):
            pltpu.make_async_copy(
                ostage.at[oslot], out_hbm.at[pl.ds(base + j, 2)],
                osem.at[oslot],
            ).start()

        def wait_out(oslot, j):
            pltpu.make_async_copy(
                ostage.at[oslot], out_hbm.at[pl.ds(base + j, 2)],
                osem.at[oslot],
            ).wait()

        start_gather(0, 0)
        start_gather(1, 1)

        @pl.loop(0, PW, step=4)
        def _(j):
            @pl.when(jnp.logical_and(jax.lax.rem(j, _IC) == _IC - 4,
                                     j + 4 < PW))
            def _():
                idx_copy(j // _IC + 1).wait()

            @pl.when(j >= 4)
            def _():
                wait_out(0, j - 4)

            wait_gather(j, 0)
            accum(0, 0, 0)
            start_gather(j + 2, 0)
            wait_gather(j + 1, 1)
            accum(1, 0, 1)
            start_gather(j + 3, 1)
            start_out(0, j)

            @pl.when(j >= 4)
            def _():
                wait_out(1, j - 2)

            wait_gather(j + 2, 0)
            accum(0, 1, 0)

            @pl.when(j + 4 < PW)
            def _():
                start_gather(j + 4, 0)

            wait_gather(j + 3, 1)
            accum(1, 1, 1)

            @pl.when(j + 5 < PW)
            def _():
                start_gather(j + 5, 1)

            start_out(1, j + 2)

            @pl.when(jnp.logical_and(jax.lax.rem(j, _IC) == _IC - 4,
                                     j < PW - 2 * _IC))
            def _():
                idx_copy(j // _IC + 2).start()

        wait_out(0, PW - 4)
        wait_out(1, PW - 2)

    return sums_kernel(x, table3)


def _fix_body(acc_ref, xv_ref, e0_ref, out_ref):
    D = out_ref.shape[1]
    SL = acc_ref.shape[1]
    n0 = jnp.sum((xv_ref[...] == 0).astype(jnp.float32), axis=1)
    for g in range(SL):
        w = min(_LT, D - g * _LT)
        if w <= 0:
            break
        out_ref[:, g * _LT:g * _LT + w] = jnp.tanh(
            acc_ref[:, g, :w] - n0 * e0_ref[:, g, :w]
        )


def _tanh_fix(sums3, x4, e03, d_out, tb):
    B, SL, LT = sums3.shape
    nt = B // tb
    L = x4.shape[1]
    return pl.pallas_call(
        _fix_body,
        grid=(nt,),
        in_specs=[
            pl.BlockSpec((tb, SL, LT), lambda t: (t, 0, 0)),
            pl.BlockSpec((tb, L, 1), lambda t: (t, 0, 0)),
            pl.BlockSpec((1, SL, LT), lambda t: (0, 0, 0)),
        ],
        out_specs=pl.BlockSpec((tb, d_out), lambda t: (t, 0)),
        out_shape=jax.ShapeDtypeStruct((B, d_out), jnp.float32),
        compiler_params=pltpu.CompilerParams(
            dimension_semantics=("arbitrary",),
        ),
    )(sums3, x4, e03)


def kernel(x, embeddings):
    xi = x.astype(jnp.int32)
    B, L = xi.shape
    V, D = embeddings.shape
    sl = pl.cdiv(D, _LT)
    dp = sl * _LT
    table3 = _repack_table(embeddings, sl)
    sums3 = _sc_sums(xi, table3, B, L)
    e03 = jnp.pad(lax.slice(embeddings, (0, 0), (1, D)),
                  ((0, 0), (0, dp - D))).reshape(1, sl, _LT)
    return _tanh_fix(sums3, xi.reshape(B, L, 1), e03, D, tb=256)


# speedup vs baseline: 2.4447x; 1.0425x over previous
import functools

import jax
import jax.numpy as jnp
from jax import lax
from jax.experimental import pallas as pl
from jax.experimental.pallas import tpu as pltpu
from jax.experimental.pallas import tpu_sc as plsc

_NC = 2
_NS = 16
_NW = _NC * _NS
_CHUNK = 16
_LT = 128
_LS = 64


def _repack_body(in_ref, out_ref, *, D):
    SL = out_ref.shape[1]
    for g in range(SL):
        w = min(_LT, D - g * _LT)
        if w <= 0:
            break
        out_ref[:, g, :w] = in_ref[:, g * _LT:g * _LT + w]


def _repack_table(embeddings, sl):
    V, D = embeddings.shape
    R = 1024
    nt = pl.cdiv(V, R)
    return pl.pallas_call(
        functools.partial(_repack_body, D=D),
        grid=(nt,),
        in_specs=[pl.BlockSpec((R, sl * _LT), lambda t: (t, 0))],
        out_specs=pl.BlockSpec((R, sl, _LT), lambda t: (t, 0, 0)),
        out_shape=jax.ShapeDtypeStruct((V, sl, _LT), jnp.float32),
        compiler_params=pltpu.CompilerParams(
            dimension_semantics=("arbitrary",),
        ),
    )(embeddings)


_IC = 32


def _sc_sums(x, table3, B, L):
    V, SL, LT = table3.shape
    PW = B // _NW
    NQ = _LT // _CHUNK
    mesh = plsc.VectorSubcoreMesh(core_axis_name="c", subcore_axis_name="s")

    @functools.partial(
        pl.kernel,
        out_type=jax.ShapeDtypeStruct((B, SL, _LT), jnp.float32),
        mesh=mesh,
        scratch_types=[
            pltpu.VMEM((2, _IC, L), jnp.int32),
            pltpu.VMEM((2, L, SL, _LT), jnp.float32),
            pltpu.VMEM((2, 2, SL, _LT), jnp.float32),
            pltpu.SemaphoreType.DMA((2,)),
            pltpu.SemaphoreType.DMA((2,)),
            pltpu.SemaphoreType.DMA,
        ],
    )
    def sums_kernel(x_hbm, table_hbm, out_hbm, idx_v, rows_v, ostage, gsem,
                    osem, isem):
        wid = lax.axis_index("s") * _NC + lax.axis_index("c")
        base = wid * PW

        def idx_copy(chunk):
            return pltpu.make_async_copy(
                x_hbm.at[pl.ds(base + chunk * _IC, _IC)],
                idx_v.at[jax.lax.rem(chunk, 2)],
                isem,
            )

        pltpu.sync_copy(x_hbm.at[pl.ds(base, _IC)], idx_v.at[0])
        idx_copy(1).start()

        def gather_copy(j, slot):
            jidx = idx_v.at[jax.lax.rem(j // _IC, 2), jax.lax.rem(j, _IC)]
            return pltpu.make_async_copy(
                table_hbm.at[jidx], rows_v.at[slot], gsem.at[slot]
            )

        def start_gather(j, slot):
            gather_copy(j, slot).start()

        def wait_gather(j, slot):
            gather_copy(j, slot).wait()

        def accum(rslot, oslot, orow):
            src = rows_v.at[rslot]

            @pl.loop(0, SL)
            def _(g):
                @pl.loop(0, NQ)
                def _(q):
                    acc = jnp.zeros((_CHUNK,), jnp.float32)
                    for r in range(L):
                        acc = acc + src[r, g, pl.ds(q * _CHUNK, _CHUNK)]
                    ostage[oslot, orow, g, pl.ds(q * _CHUNK, _CHUNK)] = acc

        def start_out(oslot, j):
            pltpu.make_async_copy(
                ostage.at[oslot], out_hbm.at[pl.ds(base + j, 2)],
                osem.at[oslot],
            ).start()

        def wait_out(oslot, j):
            pltpu.make_async_copy(
                ostage.at[oslot], out_hbm.at[pl.ds(base + j, 2)],
                osem.at[oslot],
            ).wait()

        start_gather(0, 0)
        start_gather(1, 1)

        @pl.loop(0, PW, step=4)
        def _(j):
            @pl.when(jnp.logical_and(jax.lax.rem(j, _IC) == _IC - 4,
                                     j + 4 < PW))
            def _():
                idx_copy(j // _IC + 1).wait()

            @pl.when(j >= 4)
            def _():
                wait_out(0, j - 4)

            wait_gather(j, 0)
            accum(0, 0, 0)
            start_gather(j + 2, 0)
            wait_gather(j + 1, 1)
            accum(1, 0, 1)
            start_gather(j + 3, 1)
            start_out(0, j)

            @pl.when(j >= 4)
            def _():
                wait_out(1, j - 2)

            wait_gather(j + 2, 0)
            accum(0, 1, 0)

            @pl.when(j + 4 < PW)
            def _():
                start_gather(j + 4, 0)

            wait_gather(j + 3, 1)
            accum(1, 1, 1)

            @pl.when(j + 5 < PW)
            def _():
                start_gather(j + 5, 1)

            start_out(1, j + 2)

            @pl.when(jnp.logical_and(jax.lax.rem(j, _IC) == _IC - 4,
                                     j < PW - 2 * _IC))
            def _():
                idx_copy(j // _IC + 2).start()

        wait_out(0, PW - 4)
        wait_out(1, PW - 2)

    return sums_kernel(x, table3)


def _fix_body(acc_ref, xv_ref, e0_ref, out_ref):
    D = out_ref.shape[1]
    SL = acc_ref.shape[1]
    n0 = jnp.sum((xv_ref[...] == 0).astype(jnp.float32), axis=1)
    for g in range(SL):
        w = min(_LT, D - g * _LT)
        if w <= 0:
            break
        out_ref[:, g * _LT:g * _LT + w] = jnp.tanh(
            acc_ref[:, g, :w] - n0 * e0_ref[:, g, :w]
        )


def _tanh_fix(sums3, x4, e03, d_out, tb):
    B, SL, LT = sums3.shape
    nt = B // tb
    L = x4.shape[1]
    return pl.pallas_call(
        _fix_body,
        grid=(nt,),
        in_specs=[
            pl.BlockSpec((tb, SL, LT), lambda t: (t, 0, 0)),
            pl.BlockSpec((tb, L, 1), lambda t: (t, 0, 0)),
            pl.BlockSpec((1, SL, LT), lambda t: (0, 0, 0)),
        ],
        out_specs=pl.BlockSpec((tb, d_out), lambda t: (t, 0)),
        out_shape=jax.ShapeDtypeStruct((B, d_out), jnp.float32),
        compiler_params=pltpu.CompilerParams(
            dimension_semantics=("arbitrary",),
        ),
    )(sums3, x4, e03)


def kernel(x, embeddings):
    xi = x.astype(jnp.int32)
    B, L = xi.shape
    V, D = embeddings.shape
    sl = pl.cdiv(D, _LT)
    dp = sl * _LT
    table3 = _repack_table(embeddings, sl)
    sums3 = _sc_sums(xi, table3, B, L)
    e03 = jnp.pad(lax.slice(embeddings, (0, 0), (1, D)),
                  ((0, 0), (0, dp - D))).reshape(1, sl, _LT)
    return _tanh_fix(sums3, xi.reshape(B, L, 1), e03, D, tb=256)


# speedup vs baseline: 2.5136x; 1.0282x over previous
import functools

import jax
import jax.numpy as jnp
from jax import lax
from jax.experimental import pallas as pl
from jax.experimental.pallas import tpu as pltpu
from jax.experimental.pallas import tpu_sc as plsc

_NC = 2
_NS = 16
_NW = _NC * _NS
_CHUNK = 16
_LT = 128
_LS = 64


def _repack_body(in_ref, out_ref, *, D):
    SL = out_ref.shape[1]
    for g in range(SL):
        w = min(_LT, D - g * _LT)
        if w <= 0:
            break
        out_ref[:, g, :w] = in_ref[:, g * _LT:g * _LT + w]


def _repack_table(embeddings, sl):
    V, D = embeddings.shape
    R = 1024
    nt = pl.cdiv(V, R)
    return pl.pallas_call(
        functools.partial(_repack_body, D=D),
        grid=(nt,),
        in_specs=[pl.BlockSpec((R, sl * _LT), lambda t: (t, 0))],
        out_specs=pl.BlockSpec((R, sl, _LT), lambda t: (t, 0, 0)),
        out_shape=jax.ShapeDtypeStruct((V, sl, _LT), jnp.float32),
        compiler_params=pltpu.CompilerParams(
            dimension_semantics=("arbitrary",),
        ),
    )(embeddings)


_IC = 32


def _sc_sums(x, table3, B, L):
    V, SL, LT = table3.shape
    PW = B // _NW
    NQ = _LT // _CHUNK
    mesh = plsc.VectorSubcoreMesh(core_axis_name="c", subcore_axis_name="s")

    @functools.partial(
        pl.kernel,
        out_type=jax.ShapeDtypeStruct((B, SL, _LT), jnp.float32),
        mesh=mesh,
        scratch_types=[
            pltpu.VMEM((2, _IC, L), jnp.int32),
            pltpu.VMEM((2, L, SL, _LT), jnp.float32),
            pltpu.VMEM((2, 2, SL, _LT), jnp.float32),
            pltpu.SemaphoreType.DMA((2,)),
            pltpu.SemaphoreType.DMA((2,)),
            pltpu.SemaphoreType.DMA,
        ],
    )
    def sums_kernel(x_hbm, table_hbm, out_hbm, idx_v, rows_v, ostage, gsem,
                    osem, isem):
        wid = lax.axis_index("s") * _NC + lax.axis_index("c")
        base = wid * PW

        def idx_copy(chunk):
            return pltpu.make_async_copy(
                x_hbm.at[pl.ds(base + chunk * _IC, _IC)],
                idx_v.at[jax.lax.rem(chunk, 2)],
                isem,
            )

        pltpu.sync_copy(x_hbm.at[pl.ds(base, _IC)], idx_v.at[0])
        idx_copy(1).start()

        def gather_copy(j, slot):
            jidx = idx_v.at[jax.lax.rem(j // _IC, 2), jax.lax.rem(j, _IC)]
            return pltpu.make_async_copy(
                table_hbm.at[jidx], rows_v.at[slot], gsem.at[slot]
            )

        def start_gather(j, slot):
            gather_copy(j, slot).start()

        def wait_gather(j, slot):
            gather_copy(j, slot).wait()

        def accum(rslot, oslot, orow):
            src = rows_v.at[rslot]

            @pl.loop(0, SL)
            def _(g):
                @pl.loop(0, NQ)
                def _(q):
                    acc = jnp.zeros((_CHUNK,), jnp.float32)
                    for r in range(L):
                        acc = acc + src[r, g, pl.ds(q * _CHUNK, _CHUNK)]
                    ostage[oslot, orow, g, pl.ds(q * _CHUNK, _CHUNK)] = acc

        def start_out(oslot, j):
            pltpu.make_async_copy(
                ostage.at[oslot], out_hbm.at[pl.ds(base + j, 2)],
                osem.at[oslot],
            ).start()

        def wait_out(oslot, j):
            pltpu.make_async_copy(
                ostage.at[oslot], out_hbm.at[pl.ds(base + j, 2)],
                osem.at[oslot],
            ).wait()

        start_gather(0, 0)
        start_gather(1, 1)

        @pl.loop(0, PW, step=4)
        def _(j):
            @pl.when(jnp.logical_and(jax.lax.rem(j, _IC) == _IC - 4,
                                     j + 4 < PW))
            def _():
                idx_copy(j // _IC + 1).wait()

            @pl.when(j >= 4)
            def _():
                wait_out(0, j - 4)

            wait_gather(j, 0)
            accum(0, 0, 0)
            start_gather(j + 2, 0)
            wait_gather(j + 1, 1)
            accum(1, 0, 1)
            start_gather(j + 3, 1)
            start_out(0, j)

            @pl.when(j >= 4)
            def _():
                wait_out(1, j - 2)

            wait_gather(j + 2, 0)
            accum(0, 1, 0)

            @pl.when(j + 4 < PW)
            def _():
                start_gather(j + 4, 0)

            wait_gather(j + 3, 1)
            accum(1, 1, 1)

            @pl.when(j + 5 < PW)
            def _():
                start_gather(j + 5, 1)

            start_out(1, j + 2)

            @pl.when(jnp.logical_and(jax.lax.rem(j, _IC) == _IC - 4,
                                     j < PW - 2 * _IC))
            def _():
                idx_copy(j // _IC + 2).start()

        wait_out(0, PW - 4)
        wait_out(1, PW - 2)

    return sums_kernel(x, table3)


def _fix_body(acc_ref, xv_ref, e0_ref, out_ref):
    D = out_ref.shape[1]
    SL = acc_ref.shape[1]
    n0 = jnp.sum((xv_ref[...] == 0).astype(jnp.float32), axis=1,
                 keepdims=True)
    for g in range(SL):
        w = min(_LT, D - g * _LT)
        if w <= 0:
            break
        out_ref[:, g * _LT:g * _LT + w] = jnp.tanh(
            acc_ref[:, g, :w] - n0 * e0_ref[:, g, :w]
        )


def _tanh_fix(sums3, x4, e03, d_out, tb):
    B, SL, LT = sums3.shape
    nt = B // tb
    L = x4.shape[1]
    return pl.pallas_call(
        _fix_body,
        grid=(nt,),
        in_specs=[
            pl.BlockSpec((tb, SL, LT), lambda t: (t, 0, 0)),
            pl.BlockSpec((tb, L), lambda t: (t, 0)),
            pl.BlockSpec((1, SL, LT), lambda t: (0, 0, 0)),
        ],
        out_specs=pl.BlockSpec((tb, d_out), lambda t: (t, 0)),
        out_shape=jax.ShapeDtypeStruct((B, d_out), jnp.float32),
        compiler_params=pltpu.CompilerParams(
            dimension_semantics=("arbitrary",),
        ),
    )(sums3, x4, e03)


def kernel(x, embeddings):
    xi = x.astype(jnp.int32)
    B, L = xi.shape
    V, D = embeddings.shape
    sl = pl.cdiv(D, _LT)
    dp = sl * _LT
    table3 = _repack_table(embeddings, sl)
    sums3 = _sc_sums(xi, table3, B, L)
    e03 = jnp.pad(lax.slice(embeddings, (0, 0), (1, D)),
                  ((0, 0), (0, dp - D))).reshape(1, sl, _LT)
    return _tanh_fix(sums3, xi, e03, D, tb=256)


# speedup vs baseline: 2.5167x; 1.0012x over previous
import functools

import jax
import jax.numpy as jnp
from jax import lax
from jax.experimental import pallas as pl
from jax.experimental.pallas import tpu as pltpu
from jax.experimental.pallas import tpu_sc as plsc

_NC = 2
_NS = 16
_NW = _NC * _NS
_CHUNK = 16
_LT = 128
_LS = 64


def _repack_body(in_ref, out_ref, *, D):
    SL = out_ref.shape[1]
    for g in range(SL):
        w = min(_LT, D - g * _LT)
        if w <= 0:
            break
        out_ref[:, g, :w] = in_ref[:, g * _LT:g * _LT + w]


def _repack_table(embeddings, sl):
    V, D = embeddings.shape
    R = 1024
    nt = pl.cdiv(V, R)
    return pl.pallas_call(
        functools.partial(_repack_body, D=D),
        grid=(nt,),
        in_specs=[pl.BlockSpec((R, D), lambda t: (t, 0))],
        out_specs=pl.BlockSpec((R, sl, _LT), lambda t: (t, 0, 0)),
        out_shape=jax.ShapeDtypeStruct((V, sl, _LT), jnp.float32),
        compiler_params=pltpu.CompilerParams(
            dimension_semantics=("arbitrary",),
        ),
    )(embeddings)


_IC = 32


def _sc_sums(x, table3, B, L):
    V, SL, LT = table3.shape
    PW = B // _NW
    NQ = _LT // _CHUNK
    mesh = plsc.VectorSubcoreMesh(core_axis_name="c", subcore_axis_name="s")

    @functools.partial(
        pl.kernel,
        out_type=jax.ShapeDtypeStruct((B, SL, _LT), jnp.float32),
        mesh=mesh,
        scratch_types=[
            pltpu.VMEM((2, _IC, L), jnp.int32),
            pltpu.VMEM((2, L, SL, _LT), jnp.float32),
            pltpu.VMEM((2, 2, SL, _LT), jnp.float32),
            pltpu.SemaphoreType.DMA((2,)),
            pltpu.SemaphoreType.DMA((2,)),
            pltpu.SemaphoreType.DMA,
        ],
    )
    def sums_kernel(x_hbm, table_hbm, out_hbm, idx_v, rows_v, ostage, gsem,
                    osem, isem):
        wid = lax.axis_index("s") * _NC + lax.axis_index("c")
        base = wid * PW

        def idx_copy(chunk):
            return pltpu.make_async_copy(
                x_hbm.at[pl.ds(base + chunk * _IC, _IC)],
                idx_v.at[jax.lax.rem(chunk, 2)],
                isem,
            )

        pltpu.sync_copy(x_hbm.at[pl.ds(base, _IC)], idx_v.at[0])
        idx_copy(1).start()

        def gather_copy(j, slot):
            jidx = idx_v.at[jax.lax.rem(j // _IC, 2), jax.lax.rem(j, _IC)]
            return pltpu.make_async_copy(
                table_hbm.at[jidx], rows_v.at[slot], gsem.at[slot]
            )

        def start_gather(j, slot):
            gather_copy(j, slot).start()

        def wait_gather(j, slot):
            gather_copy(j, slot).wait()

        def accum(rslot, oslot, orow):
            src = rows_v.at[rslot]

            @pl.loop(0, SL)
            def _(g):
                @pl.loop(0, NQ)
                def _(q):
                    acc = jnp.zeros((_CHUNK,), jnp.float32)
                    for r in range(L):
                        acc = acc + src[r, g, pl.ds(q * _CHUNK, _CHUNK)]
                    ostage[oslot, orow, g, pl.ds(q * _CHUNK, _CHUNK)] = acc

        def start_out(oslot, j):
            pltpu.make_async_copy(
                ostage.at[oslot], out_hbm.at[pl.ds(base + j, 2)],
                osem.at[oslot],
            ).start()

        def wait_out(oslot, j):
            pltpu.make_async_copy(
                ostage.at[oslot], out_hbm.at[pl.ds(base + j, 2)],
                osem.at[oslot],
            ).wait()

        start_gather(0, 0)
        start_gather(1, 1)

        @pl.loop(0, PW, step=4)
        def _(j):
            @pl.when(jnp.logical_and(jax.lax.rem(j, _IC) == _IC - 4,
                                     j + 4 < PW))
            def _():
                idx_copy(j // _IC + 1).wait()

            @pl.when(j >= 4)
            def _():
                wait_out(0, j - 4)

            wait_gather(j, 0)
            accum(0, 0, 0)
            start_gather(j + 2, 0)
            wait_gather(j + 1, 1)
            accum(1, 0, 1)
            start_gather(j + 3, 1)
            start_out(0, j)

            @pl.when(j >= 4)
            def _():
                wait_out(1, j - 2)

            wait_gather(j + 2, 0)
            accum(0, 1, 0)

            @pl.when(j + 4 < PW)
            def _():
                start_gather(j + 4, 0)

            wait_gather(j + 3, 1)
            accum(1, 1, 1)

            @pl.when(j + 5 < PW)
            def _():
                start_gather(j + 5, 1)

            start_out(1, j + 2)

            @pl.when(jnp.logical_and(jax.lax.rem(j, _IC) == _IC - 4,
                                     j < PW - 2 * _IC))
            def _():
                idx_copy(j // _IC + 2).start()

        wait_out(0, PW - 4)
        wait_out(1, PW - 2)

    return sums_kernel(x, table3)


def _fix_body(acc_ref, xv_ref, e0_ref, out_ref):
    D = out_ref.shape[1]
    SL = acc_ref.shape[1]
    n0 = jnp.sum((xv_ref[...] == 0).astype(jnp.float32), axis=1,
                 keepdims=True)
    for g in range(SL):
        w = min(_LT, D - g * _LT)
        if w <= 0:
            break
        out_ref[:, g * _LT:g * _LT + w] = jnp.tanh(
            acc_ref[:, g, :w] - n0 * e0_ref[:, g, :w]
        )


def _tanh_fix(sums3, x4, e03, d_out, tb):
    B, SL, LT = sums3.shape
    nt = B // tb
    L = x4.shape[1]
    return pl.pallas_call(
        _fix_body,
        grid=(nt,),
        in_specs=[
            pl.BlockSpec((tb, SL, LT), lambda t: (t, 0, 0)),
            pl.BlockSpec((tb, L), lambda t: (t, 0)),
            pl.BlockSpec((1, SL, LT), lambda t: (0, 0, 0)),
        ],
        out_specs=pl.BlockSpec((tb, d_out), lambda t: (t, 0)),
        out_shape=jax.ShapeDtypeStruct((B, d_out), jnp.float32),
        compiler_params=pltpu.CompilerParams(
            dimension_semantics=("arbitrary",),
        ),
    )(sums3, x4, e03)


def kernel(x, embeddings):
    xi = x.astype(jnp.int32)
    B, L = xi.shape
    V, D = embeddings.shape
    sl = pl.cdiv(D, _LT)
    dp = sl * _LT
    table3 = _repack_table(embeddings, sl)
    sums3 = _sc_sums(xi, table3, B, L)
    e03 = jnp.pad(lax.slice(embeddings, (0, 0), (1, D)),
                  ((0, 0), (0, dp - D))).reshape(1, sl, _LT)
    return _tanh_fix(sums3, xi, e03, D, tb=256)


# speedup vs baseline: 2.5182x; 1.0006x over previous
import functools

import jax
import jax.numpy as jnp
from jax import lax
from jax.experimental import pallas as pl
from jax.experimental.pallas import tpu as pltpu
from jax.experimental.pallas import tpu_sc as plsc

_NC = 2
_NS = 16
_NW = _NC * _NS
_CHUNK = 16
_LT = 128


def _repack_body(in_ref, out_ref, *, D):
    SL = out_ref.shape[1]
    for g in range(SL):
        w = min(_LT, D - g * _LT)
        if w <= 0:
            break
        out_ref[:, g, :w] = in_ref[:, g * _LT:g * _LT + w]


def _repack_table(embeddings, sl):
    V, D = embeddings.shape
    R = 1024
    nt = pl.cdiv(V, R)
    return pl.pallas_call(
        functools.partial(_repack_body, D=D),
        grid=(nt,),
        in_specs=[pl.BlockSpec((R, D), lambda t: (t, 0))],
        out_specs=pl.BlockSpec((R, sl, _LT), lambda t: (t, 0, 0)),
        out_shape=jax.ShapeDtypeStruct((V, sl, _LT), jnp.float32),
        compiler_params=pltpu.CompilerParams(
            dimension_semantics=("arbitrary",),
        ),
    )(embeddings)


_IC = 32


def _sc_sums(x, table3, B, L):
    V, SL, LT = table3.shape
    PW = B // _NW
    NQ = _LT // _CHUNK
    mesh = plsc.VectorSubcoreMesh(core_axis_name="c", subcore_axis_name="s")

    @functools.partial(
        pl.kernel,
        out_type=jax.ShapeDtypeStruct((B, SL, _LT), jnp.float32),
        mesh=mesh,
        scratch_types=[
            pltpu.VMEM((2, _IC, L), jnp.int32),
            pltpu.VMEM((2, L, SL, _LT), jnp.float32),
            pltpu.VMEM((2, 2, SL, _LT), jnp.float32),
            pltpu.SemaphoreType.DMA((2,)),
            pltpu.SemaphoreType.DMA((2,)),
            pltpu.SemaphoreType.DMA,
        ],
    )
    def sums_kernel(x_hbm, table_hbm, out_hbm, idx_v, rows_v, ostage, gsem,
                    osem, isem):
        wid = lax.axis_index("s") * _NC + lax.axis_index("c")
        base = wid * PW

        def idx_copy(chunk):
            return pltpu.make_async_copy(
                x_hbm.at[pl.ds(base + chunk * _IC, _IC)],
                idx_v.at[jax.lax.rem(chunk, 2)],
                isem,
            )

        pltpu.sync_copy(x_hbm.at[pl.ds(base, _IC)], idx_v.at[0])
        idx_copy(1).start()

        def gather_copy(j, slot):
            jidx = idx_v.at[jax.lax.rem(j // _IC, 2), jax.lax.rem(j, _IC)]
            return pltpu.make_async_copy(
                table_hbm.at[jidx], rows_v.at[slot], gsem.at[slot]
            )

        def start_gather(j, slot):
            gather_copy(j, slot).start()

        def wait_gather(j, slot):
            gather_copy(j, slot).wait()

        def accum(rslot, oslot, orow):
            src = rows_v.at[rslot]

            @pl.loop(0, SL)
            def _(g):
                @pl.loop(0, NQ)
                def _(q):
                    acc = jnp.zeros((_CHUNK,), jnp.float32)
                    for r in range(L):
                        acc = acc + src[r, g, pl.ds(q * _CHUNK, _CHUNK)]
                    ostage[oslot, orow, g, pl.ds(q * _CHUNK, _CHUNK)] = acc

        def start_out(oslot, j):
            pltpu.make_async_copy(
                ostage.at[oslot], out_hbm.at[pl.ds(base + j, 2)],
                osem.at[oslot],
            ).start()

        def wait_out(oslot, j):
            pltpu.make_async_copy(
                ostage.at[oslot], out_hbm.at[pl.ds(base + j, 2)],
                osem.at[oslot],
            ).wait()

        start_gather(0, 0)
        start_gather(1, 1)

        @pl.loop(0, PW, step=4)
        def _(j):
            @pl.when(jnp.logical_and(jax.lax.rem(j, _IC) == _IC - 4,
                                     j + 4 < PW))
            def _():
                idx_copy(j // _IC + 1).wait()

            @pl.when(j >= 4)
            def _():
                wait_out(0, j - 4)

            wait_gather(j, 0)
            accum(0, 0, 0)
            start_gather(j + 2, 0)
            wait_gather(j + 1, 1)
            accum(1, 0, 1)
            start_gather(j + 3, 1)
            start_out(0, j)

            @pl.when(j >= 4)
            def _():
                wait_out(1, j - 2)

            wait_gather(j + 2, 0)
            accum(0, 1, 0)

            @pl.when(j + 4 < PW)
            def _():
                start_gather(j + 4, 0)

            wait_gather(j + 3, 1)
            accum(1, 1, 1)

            @pl.when(j + 5 < PW)
            def _():
                start_gather(j + 5, 1)

            start_out(1, j + 2)

            @pl.when(jnp.logical_and(jax.lax.rem(j, _IC) == _IC - 4,
                                     j < PW - 2 * _IC))
            def _():
                idx_copy(j // _IC + 2).start()

        wait_out(0, PW - 4)
        wait_out(1, PW - 2)

    return sums_kernel(x, table3)


def _fix_body(acc_ref, xv_ref, e0_ref, out_ref):
    D = out_ref.shape[1]
    SL = acc_ref.shape[1]
    n0 = jnp.sum((xv_ref[...] == 0).astype(jnp.float32), axis=1,
                 keepdims=True)
    for g in range(SL):
        w = min(_LT, D - g * _LT)
        if w <= 0:
            break
        out_ref[:, g * _LT:g * _LT + w] = jnp.tanh(
            acc_ref[:, g, :w] - n0 * e0_ref[:, g, :w]
        )


def _tanh_fix(sums3, x4, e03, d_out, tb):
    B, SL, LT = sums3.shape
    nt = B // tb
    L = x4.shape[1]
    return pl.pallas_call(
        _fix_body,
        grid=(nt,),
        in_specs=[
            pl.BlockSpec((tb, SL, LT), lambda t: (t, 0, 0)),
            pl.BlockSpec((tb, L), lambda t: (t, 0)),
            pl.BlockSpec((1, SL, LT), lambda t: (0, 0, 0)),
        ],
        out_specs=pl.BlockSpec((tb, d_out), lambda t: (t, 0)),
        out_shape=jax.ShapeDtypeStruct((B, d_out), jnp.float32),
        compiler_params=pltpu.CompilerParams(
            dimension_semantics=("arbitrary",),
        ),
    )(sums3, x4, e03)


def kernel(x, embeddings):
    xi = x.astype(jnp.int32)
    B, L = xi.shape
    V, D = embeddings.shape
    sl = pl.cdiv(D, _LT)
    dp = sl * _LT
    table3 = _repack_table(embeddings, sl)
    sums3 = _sc_sums(xi, table3, B, L)
    e03 = jnp.pad(lax.slice(embeddings, (0, 0), (1, D)),
                  ((0, 0), (0, dp - D))).reshape(1, sl, _LT)
    return _tanh_fix(sums3, xi, e03, D, tb=256)


# speedup vs baseline: 2.5793x; 1.0243x over previous
import functools

import jax
import jax.numpy as jnp
from jax import lax
from jax.experimental import pallas as pl
from jax.experimental.pallas import tpu as pltpu
from jax.experimental.pallas import tpu_sc as plsc

_NC = 2
_NS = 16
_NW = _NC * _NS
_CHUNK = 16
_LT = 128


def _repack_body(in_ref, out_ref, *, D):
    SL = out_ref.shape[1]
    for g in range(SL):
        w = min(_LT, D - g * _LT)
        if w <= 0:
            break
        out_ref[:, g, :w] = in_ref[:, g * _LT:g * _LT + w]


def _repack_table(embeddings, sl):
    V, D = embeddings.shape
    R = 2048
    nt = pl.cdiv(V, R)
    return pl.pallas_call(
        functools.partial(_repack_body, D=D),
        grid=(nt,),
        in_specs=[pl.BlockSpec((R, D), lambda t: (t, 0))],
        out_specs=pl.BlockSpec((R, sl, _LT), lambda t: (t, 0, 0)),
        out_shape=jax.ShapeDtypeStruct((V, sl, _LT), jnp.float32),
        compiler_params=pltpu.CompilerParams(
            dimension_semantics=("arbitrary",),
        ),
    )(embeddings)


_IC = 32


def _sc_sums(x, table3, B, L):
    V, SL, LT = table3.shape
    PW = B // _NW
    NQ = _LT // _CHUNK
    mesh = plsc.VectorSubcoreMesh(core_axis_name="c", subcore_axis_name="s")

    @functools.partial(
        pl.kernel,
        out_type=jax.ShapeDtypeStruct((B, SL, _LT), jnp.float32),
        mesh=mesh,
        scratch_types=[
            pltpu.VMEM((2, _IC, L), jnp.int32),
            pltpu.VMEM((2, L, SL, _LT), jnp.float32),
            pltpu.VMEM((2, 2, SL, _LT), jnp.float32),
            pltpu.SemaphoreType.DMA((2,)),
            pltpu.SemaphoreType.DMA((2,)),
            pltpu.SemaphoreType.DMA,
        ],
    )
    def sums_kernel(x_hbm, table_hbm, out_hbm, idx_v, rows_v, ostage, gsem,
                    osem, isem):
        wid = lax.axis_index("s") * _NC + lax.axis_index("c")
        base = wid * PW

        def idx_copy(chunk):
            return pltpu.make_async_copy(
                x_hbm.at[pl.ds(base + chunk * _IC, _IC)],
                idx_v.at[jax.lax.rem(chunk, 2)],
                isem,
            )

        pltpu.sync_copy(x_hbm.at[pl.ds(base, _IC)], idx_v.at[0])
        idx_copy(1).start()

        def gather_copy(j, slot):
            jidx = idx_v.at[jax.lax.rem(j // _IC, 2), jax.lax.rem(j, _IC)]
            return pltpu.make_async_copy(
                table_hbm.at[jidx], rows_v.at[slot], gsem.at[slot]
            )

        def start_gather(j, slot):
            gather_copy(j, slot).start()

        def wait_gather(j, slot):
            gather_copy(j, slot).wait()

        def accum(rslot, oslot, orow):
            src = rows_v.at[rslot]

            @pl.loop(0, SL)
            def _(g):
                @pl.loop(0, NQ)
                def _(q):
                    acc = jnp.zeros((_CHUNK,), jnp.float32)
                    for r in range(L):
                        acc = acc + src[r, g, pl.ds(q * _CHUNK, _CHUNK)]
                    ostage[oslot, orow, g, pl.ds(q * _CHUNK, _CHUNK)] = acc

        def start_out(oslot, j):
            pltpu.make_async_copy(
                ostage.at[oslot], out_hbm.at[pl.ds(base + j, 2)],
                osem.at[oslot],
            ).start()

        def wait_out(oslot, j):
            pltpu.make_async_copy(
                ostage.at[oslot], out_hbm.at[pl.ds(base + j, 2)],
                osem.at[oslot],
            ).wait()

        start_gather(0, 0)
        start_gather(1, 1)

        @pl.loop(0, PW, step=4)
        def _(j):
            @pl.when(jnp.logical_and(jax.lax.rem(j, _IC) == _IC - 4,
                                     j + 4 < PW))
            def _():
                idx_copy(j // _IC + 1).wait()

            @pl.when(j >= 4)
            def _():
                wait_out(0, j - 4)

            wait_gather(j, 0)
            accum(0, 0, 0)
            start_gather(j + 2, 0)
            wait_gather(j + 1, 1)
            accum(1, 0, 1)
            start_gather(j + 3, 1)
            start_out(0, j)

            @pl.when(j >= 4)
            def _():
                wait_out(1, j - 2)

            wait_gather(j + 2, 0)
            accum(0, 1, 0)

            @pl.when(j + 4 < PW)
            def _():
                start_gather(j + 4, 0)

            wait_gather(j + 3, 1)
            accum(1, 1, 1)

            @pl.when(j + 5 < PW)
            def _():
                start_gather(j + 5, 1)

            start_out(1, j + 2)

            @pl.when(jnp.logical_and(jax.lax.rem(j, _IC) == _IC - 4,
                                     j < PW - 2 * _IC))
            def _():
                idx_copy(j // _IC + 2).start()

        wait_out(0, PW - 4)
        wait_out(1, PW - 2)

    return sums_kernel(x, table3)


def _fix_body(acc_ref, xv_ref, e0_ref, out_ref):
    D = out_ref.shape[1]
    SL = acc_ref.shape[1]
    n0 = jnp.sum((xv_ref[...] == 0).astype(jnp.float32), axis=1,
                 keepdims=True)
    for g in range(SL):
        w = min(_LT, D - g * _LT)
        if w <= 0:
            break
        out_ref[:, g * _LT:g * _LT + w] = jnp.tanh(
            acc_ref[:, g, :w] - n0 * e0_ref[:, g, :w]
        )


def _tanh_fix(sums3, x4, e03, d_out, tb):
    B, SL, LT = sums3.shape
    nt = B // tb
    L = x4.shape[1]
    return pl.pallas_call(
        _fix_body,
        grid=(nt,),
        in_specs=[
            pl.BlockSpec((tb, SL, LT), lambda t: (t, 0, 0)),
            pl.BlockSpec((tb, L), lambda t: (t, 0)),
            pl.BlockSpec((1, SL, LT), lambda t: (0, 0, 0)),
        ],
        out_specs=pl.BlockSpec((tb, d_out), lambda t: (t, 0)),
        out_shape=jax.ShapeDtypeStruct((B, d_out), jnp.float32),
        compiler_params=pltpu.CompilerParams(
            dimension_semantics=("arbitrary",),
        ),
    )(sums3, x4, e03)


def kernel(x, embeddings):
    xi = x.astype(jnp.int32)
    B, L = xi.shape
    V, D = embeddings.shape
    sl = pl.cdiv(D, _LT)
    dp = sl * _LT
    table3 = _repack_table(embeddings, sl)
    sums3 = _sc_sums(xi, table3, B, L)
    e03 = jnp.pad(lax.slice(embeddings, (0, 0), (1, D)),
                  ((0, 0), (0, dp - D))).reshape(1, sl, _LT)
    return _tanh_fix(sums3, xi, e03, D, tb=256)


# speedup vs baseline: 2.5930x; 1.0053x over previous
import functools

import jax
import jax.numpy as jnp
from jax import lax
from jax.experimental import pallas as pl
from jax.experimental.pallas import tpu as pltpu
from jax.experimental.pallas import tpu_sc as plsc

_NC = 2
_NS = 16
_NW = _NC * _NS
_CHUNK = 16
_LT = 128


def _repack_body(in_ref, out_ref, *, D):
    SL = out_ref.shape[1]
    for g in range(SL):
        w = min(_LT, D - g * _LT)
        if w <= 0:
            break
        out_ref[:, g, :w] = in_ref[:, g * _LT:g * _LT + w]


def _repack_table(embeddings, sl):
    V, D = embeddings.shape
    R = 3072
    nt = pl.cdiv(V, R)
    return pl.pallas_call(
        functools.partial(_repack_body, D=D),
        grid=(nt,),
        in_specs=[pl.BlockSpec((R, D), lambda t: (t, 0))],
        out_specs=pl.BlockSpec((R, sl, _LT), lambda t: (t, 0, 0)),
        out_shape=jax.ShapeDtypeStruct((V, sl, _LT), jnp.float32),
        compiler_params=pltpu.CompilerParams(
            dimension_semantics=("arbitrary",),
        ),
    )(embeddings)


_IC = 32


def _sc_sums(x, table3, B, L):
    V, SL, LT = table3.shape
    PW = B // _NW
    NQ = _LT // _CHUNK
    mesh = plsc.VectorSubcoreMesh(core_axis_name="c", subcore_axis_name="s")

    @functools.partial(
        pl.kernel,
        out_type=jax.ShapeDtypeStruct((B, SL, _LT), jnp.float32),
        mesh=mesh,
        scratch_types=[
            pltpu.VMEM((2, _IC, L), jnp.int32),
            pltpu.VMEM((2, L, SL, _LT), jnp.float32),
            pltpu.VMEM((2, 2, SL, _LT), jnp.float32),
            pltpu.SemaphoreType.DMA((2,)),
            pltpu.SemaphoreType.DMA((2,)),
            pltpu.SemaphoreType.DMA,
        ],
    )
    def sums_kernel(x_hbm, table_hbm, out_hbm, idx_v, rows_v, ostage, gsem,
                    osem, isem):
        wid = lax.axis_index("s") * _NC + lax.axis_index("c")
        base = wid * PW

        def idx_copy(chunk):
            return pltpu.make_async_copy(
                x_hbm.at[pl.ds(base + chunk * _IC, _IC)],
                idx_v.at[jax.lax.rem(chunk, 2)],
                isem,
            )

        pltpu.sync_copy(x_hbm.at[pl.ds(base, _IC)], idx_v.at[0])
        idx_copy(1).start()

        def gather_copy(j, slot):
            jidx = idx_v.at[jax.lax.rem(j // _IC, 2), jax.lax.rem(j, _IC)]
            return pltpu.make_async_copy(
                table_hbm.at[jidx], rows_v.at[slot], gsem.at[slot]
            )

        def start_gather(j, slot):
            gather_copy(j, slot).start()

        def wait_gather(j, slot):
            gather_copy(j, slot).wait()

        def accum(rslot, oslot, orow):
            src = rows_v.at[rslot]

            @pl.loop(0, SL)
            def _(g):
                @pl.loop(0, NQ)
                def _(q):
                    acc = jnp.zeros((_CHUNK,), jnp.float32)
                    for r in range(L):
                        acc = acc + src[r, g, pl.ds(q * _CHUNK, _CHUNK)]
                    ostage[oslot, orow, g, pl.ds(q * _CHUNK, _CHUNK)] = acc

        def start_out(oslot, j):
            pltpu.make_async_copy(
                ostage.at[oslot], out_hbm.at[pl.ds(base + j, 2)],
                osem.at[oslot],
            ).start()

        def wait_out(oslot, j):
            pltpu.make_async_copy(
                ostage.at[oslot], out_hbm.at[pl.ds(base + j, 2)],
                osem.at[oslot],
            ).wait()

        start_gather(0, 0)
        start_gather(1, 1)

        @pl.loop(0, PW, step=4)
        def _(j):
            @pl.when(jnp.logical_and(jax.lax.rem(j, _IC) == _IC - 4,
                                     j + 4 < PW))
            def _():
                idx_copy(j // _IC + 1).wait()

            @pl.when(j >= 4)
            def _():
                wait_out(0, j - 4)

            wait_gather(j, 0)
            accum(0, 0, 0)
            start_gather(j + 2, 0)
            wait_gather(j + 1, 1)
            accum(1, 0, 1)
            start_gather(j + 3, 1)
            start_out(0, j)

            @pl.when(j >= 4)
            def _():
                wait_out(1, j - 2)

            wait_gather(j + 2, 0)
            accum(0, 1, 0)

            @pl.when(j + 4 < PW)
            def _():
                start_gather(j + 4, 0)

            wait_gather(j + 3, 1)
            accum(1, 1, 1)

            @pl.when(j + 5 < PW)
            def _():
                start_gather(j + 5, 1)

            start_out(1, j + 2)

            @pl.when(jnp.logical_and(jax.lax.rem(j, _IC) == _IC - 4,
                                     j < PW - 2 * _IC))
            def _():
                idx_copy(j // _IC + 2).start()

        wait_out(0, PW - 4)
        wait_out(1, PW - 2)

    return sums_kernel(x, table3)


def _fix_body(acc_ref, xv_ref, e0_ref, out_ref):
    D = out_ref.shape[1]
    SL = acc_ref.shape[1]
    n0 = jnp.sum((xv_ref[...] == 0).astype(jnp.float32), axis=1,
                 keepdims=True)
    for g in range(SL):
        w = min(_LT, D - g * _LT)
        if w <= 0:
            break
        out_ref[:, g * _LT:g * _LT + w] = jnp.tanh(
            acc_ref[:, g, :w] - n0 * e0_ref[:, g, :w]
        )


def _tanh_fix(sums3, x4, e03, d_out, tb):
    B, SL, LT = sums3.shape
    nt = B // tb
    L = x4.shape[1]
    return pl.pallas_call(
        _fix_body,
        grid=(nt,),
        in_specs=[
            pl.BlockSpec((tb, SL, LT), lambda t: (t, 0, 0)),
            pl.BlockSpec((tb, L), lambda t: (t, 0)),
            pl.BlockSpec((1, SL, LT), lambda t: (0, 0, 0)),
        ],
        out_specs=pl.BlockSpec((tb, d_out), lambda t: (t, 0)),
        out_shape=jax.ShapeDtypeStruct((B, d_out), jnp.float32),
        compiler_params=pltpu.CompilerParams(
            dimension_semantics=("arbitrary",),
        ),
    )(sums3, x4, e03)


def kernel(x, embeddings):
    xi = x.astype(jnp.int32)
    B, L = xi.shape
    V, D = embeddings.shape
    sl = pl.cdiv(D, _LT)
    dp = sl * _LT
    table3 = _repack_table(embeddings, sl)
    sums3 = _sc_sums(xi, table3, B, L)
    e03 = jnp.pad(lax.slice(embeddings, (0, 0), (1, D)),
                  ((0, 0), (0, dp - D))).reshape(1, sl, _LT)
    return _tanh_fix(sums3, xi, e03, D, tb=256)
